# baseline (speedup 1.0000x reference)
import jax
import jax.numpy as jnp
from jax import lax
from jax.experimental import pallas as pl
from jax.experimental.pallas import tpu as pltpu
from jax.experimental.pallas import tpu_sc as plsc

EPS_ = 1e-5
NC_, NS_, L_ = 2, 16, 16
BS_, C_, N_ = 16, 64, 32768
NH_ = N_ // 2
NB1_ = 512
CAP_ = 17440


def _sort_body(split_hbm, idxout_hbm, scores_v, kka, kkb, kia, kib, hist,
               b2s):
    wid = lax.axis_index("s") * NC_ + lax.axis_index("c")
    lane = jnp.arange(L_, dtype=jnp.int32)
    zeros16 = jnp.zeros((L_,), jnp.int32)
    ones16 = jnp.ones((L_,), jnp.int32)
    nb1f = jnp.float32(NB1_)

    @pl.when(wid < BS_)
    def _():
        b = wid
        pltpu.sync_copy(split_hbm.at[pl.ds(b, 1)], scores_v)
        sf = scores_v.at[0]

        def zero_hist(nwords):
            def z(i, _):
                hist[pl.ds(i * L_, L_)] = zeros16
                return 0
            lax.fori_loop(0, nwords // L_, z, 0, unroll=8)

        def bin1_of(s):
            return jnp.clip((s * nb1f).astype(jnp.int32), 0, NB1_ - 1)

        zero_hist(NB1_ * L_)

        def h1(j, _):
            s = sf[pl.ds(j * L_, L_)]
            plsc.addupdate_scatter(hist, [bin1_of(s) * L_ + lane], ones16)
            return 0
        lax.fori_loop(0, N_ // L_, h1, 0, unroll=8)

        def scan1(i, carry):
            cum, b1, a1, c1 = carry
            binv = NB1_ - 1 - i
            cnt = jnp.sum(hist[pl.ds(binv * L_, L_)])
            newcum = cum + cnt
            hit = (cum < NH_) & (newcum >= NH_)
            return (newcum,
                    jnp.where(hit, binv, b1),
                    jnp.where(hit, cum, a1),
                    jnp.where(hit, cnt, c1))
        _, b1, a1, c1 = lax.fori_loop(
            0, NB1_, scan1,
            (jnp.int32(0), jnp.int32(0), jnp.int32(0), jnp.int32(0)),
            unroll=4)
        b1f = b1.astype(jnp.float32)

        def bin2_of(s):
            t = s * nb1f - b1f
            return jnp.clip((t * nb1f).astype(jnp.int32), 0, NB1_ - 1)

        b2s[0] = jnp.int32(0)

        @pl.when(a1 + c1 > CAP_ - L_)
        def _():
            zero_hist(NB1_ * L_)

            def h2(j, _):
                s = sf[pl.ds(j * L_, L_)]
                m = bin1_of(s) == b1
                plsc.addupdate_scatter(hist, [bin2_of(s) * L_ + lane],
                                       ones16, mask=m)
                return 0
            lax.fori_loop(0, N_ // L_, h2, 0, unroll=8)

            def scan2(i, carry):
                cum, b2 = carry
                binv = NB1_ - 1 - i
                cnt = jnp.sum(hist[pl.ds(binv * L_, L_)])
                newcum = cum + cnt
                hit = (a1 + cum < NH_) & (a1 + newcum >= NH_)
                return newcum, jnp.where(hit, binv, b2)
            _, b2v = lax.fori_loop(0, NB1_, scan2,
                                   (jnp.int32(0), jnp.int32(0)), unroll=4)
            b2s[0] = b2v

        b2 = b2s[0]

        def comp(j, w):
            s = sf[pl.ds(j * L_, L_)]
            key = ~plsc.bitcast(s, jnp.int32)
            idxv = j * L_ + lane
            bb1 = bin1_of(s)
            keep = (bb1 > b1) | ((bb1 == b1) & (bin2_of(s) >= b2))
            plsc.store_compressed(kka.at[pl.ds(w, L_)], key, mask=keep)
            plsc.store_compressed(kia.at[pl.ds(w, L_)], idxv, mask=keep)
            return w + jnp.sum(keep.astype(jnp.int32))
        kcnt = lax.fori_loop(0, N_ // L_, comp, jnp.int32(0), unroll=4)
        minus1 = jnp.full((L_,), -1, jnp.int32)

        def padp(t, _):
            w = kcnt + t * L_

            @pl.when(w < CAP_)
            def _():
                kka[pl.ds(w, L_)] = minus1
                kia[pl.ds(w, L_)] = zeros16
            return 0
        lax.fori_loop(0, (CAP_ - NH_) // L_ + 1, padp, 0, unroll=4)
        chunk = CAP_ // L_

        HB_ = 256 * L_

        def zero_region(off):
            def z(i, _):
                hist[pl.ds(off + i * L_, L_)] = zeros16
                return 0
            lax.fori_loop(0, 256, z, 0, unroll=8)

        zero_region(0)

        def hh(v, _):
            keyv = plsc.load_gather(kka, [lane * chunk + v])
            d = keyv & 255
            plsc.addupdate_scatter(hist, [d * L_ + lane], ones16)
            return 0
        lax.fori_loop(0, chunk, hh, 0, unroll=8)

        bufs = [(kka, kia, kkb, kib), (kkb, kib, kka, kia)]
        for p in range(4):
            src_k, src_i, dst_k, dst_i = bufs[p % 2]
            sh = jnp.int32(8 * p)
            sh2 = jnp.int32(8 * (p + 1))
            off = (p % 2) * HB_
            off2 = ((p + 1) % 2) * HB_

            def pf(i, carry):
                h16 = hist[pl.ds(off + i * L_, L_)]
                exc = plsc.cumsum(h16) - h16
                hist[pl.ds(off + i * L_, L_)] = exc + carry
                return carry + jnp.sum(h16)
            lax.fori_loop(0, 256, pf, jnp.int32(0), unroll=4)
            if p < 3:
                zero_region(off2)

            def pm(v, _):
                addr = lane * chunk + v
                keyv = plsc.load_gather(src_k, [addr])
                iv = plsc.load_gather(src_i, [addr])
                d = lax.shift_right_logical(keyv, sh) & 255
                ha = off + d * L_ + lane
                pos = plsc.load_gather(hist, [ha])
                plsc.store_scatter(dst_k, [pos], keyv)
                plsc.store_scatter(dst_i, [pos], iv)
                plsc.addupdate_scatter(hist, [ha], ones16)
                if p < 3:
                    d2 = lax.shift_right_logical(keyv, sh2) & 255
                    lane2 = pos // chunk
                    plsc.addupdate_scatter(hist, [off2 + d2 * L_ + lane2],
                                           ones16)
                return 0
            lax.fori_loop(0, chunk, pm, 0, unroll=4)

        pltpu.sync_copy(kia.at[pl.ds(0, NH_)],
                        idxout_hbm.at[pl.ds(b * NH_, NH_)])


def _sc_sort(split):
    return pl.kernel(
        _sort_body,
        out_type=jax.ShapeDtypeStruct((BS_ * NH_,), jnp.int32),
        mesh=plsc.VectorSubcoreMesh(core_axis_name="c", subcore_axis_name="s"),
        compiler_params=pltpu.CompilerParams(needs_layout_passes=False),
        scratch_types=[
            pltpu.VMEM((1, N_), jnp.float32),
            pltpu.VMEM((CAP_ + L_,), jnp.int32),
            pltpu.VMEM((CAP_ + L_,), jnp.int32),
            pltpu.VMEM((CAP_ + L_,), jnp.int32),
            pltpu.VMEM((CAP_ + L_,), jnp.int32),
            pltpu.VMEM((NB1_ * L_,), jnp.int32),
            pltpu.SMEM((1,), jnp.int32),
        ],
    )(split)


def _gather_body(x_hbm, split_hbm, idx_hbm, out_hbm, idx_v,
                 row_a, row_b, out_a, out_b, isem, osem):
    wid = lax.axis_index("s") * NC_ + lax.axis_index("c")
    b = wid // 2
    half = wid % 2
    pltpu.sync_copy(idx_hbm.at[pl.ds(b * NH_, NH_)], idx_v)

    rows = [row_a, row_b]
    outs = [out_a, out_b]

    def gather_into(row_v, out_v):
        row_f, out_f = row_v.at[0], out_v.at[0]

        @plsc.parallel_loop(0, NH_ // L_, 1, unroll=8)
        def _(j):
            iv = idx_v[pl.ds(j * L_, L_)]
            out_f[pl.ds(j * L_, L_)] = plsc.load_gather(row_f, [iv])

    def src_of(i):
        return x_hbm.at[b, pl.ds(half + 2 * i, 1)]

    def dst_of(i):
        return out_hbm.at[b, pl.ds(half + 2 * i, 1)]

    nrows = C_ // 2
    in_descs = [None, None]
    out_descs = [None, None]
    in_descs[0] = pltpu.async_copy(src_of(0), rows[0], isem)
    for i in range(nrows):
        pp = i % 2
        if i + 1 < nrows:
            in_descs[(i + 1) % 2] = pltpu.async_copy(
                src_of(i + 1), rows[(i + 1) % 2], isem)
        in_descs[pp].wait()
        if out_descs[pp] is not None:
            out_descs[pp].wait()
        gather_into(rows[pp], outs[pp])
        out_descs[pp] = pltpu.async_copy(outs[pp], dst_of(i), osem)
    for d in out_descs:
        if d is not None:
            d.wait()

    @pl.when(half == 0)
    def _():
        pltpu.sync_copy(split_hbm.at[pl.ds(b, 1)], row_a)
        gather_into(row_a, out_a)
        pltpu.sync_copy(out_a, out_hbm.at[b, pl.ds(C_, 1)])


def _sc_gather(x, split, idx):
    return pl.kernel(
        _gather_body,
        out_type=jax.ShapeDtypeStruct((BS_, C_ + 1, NH_), jnp.float32),
        mesh=plsc.VectorSubcoreMesh(core_axis_name="c", subcore_axis_name="s"),
        compiler_params=pltpu.CompilerParams(needs_layout_passes=False),
        scratch_types=[
            pltpu.VMEM((NH_,), jnp.int32),
            pltpu.VMEM((1, N_), jnp.float32),
            pltpu.VMEM((1, N_), jnp.float32),
            pltpu.VMEM((1, NH_), jnp.float32),
            pltpu.VMEM((1, NH_), jnp.float32),
            pltpu.SemaphoreType.DMA,
            pltpu.SemaphoreType.DMA,
        ],
    )(x, split, idx)


def kernel(x, gamma, beta, conv_w, conv_b):
    mean = jnp.mean(x, axis=(0, 2), keepdims=True)
    var = jnp.var(x, axis=(0, 2), keepdims=True)
    h = (x - mean) / jnp.sqrt(var + EPS_)
    h = h * gamma[None, :, None] + beta[None, :, None]
    h = jnp.maximum(h, 0.0)
    logits = jnp.einsum('bcn,c->bn', h, conv_w) + conv_b[0]
    split = jax.nn.sigmoid(logits)
    idx_flat = _sc_sort(split)
    return _sc_gather(x, split, idx_flat)


# speedup vs baseline: 1.2075x; 1.2075x over previous
import jax
import jax.numpy as jnp
from jax import lax
from jax.experimental import pallas as pl
from jax.experimental.pallas import tpu as pltpu
from jax.experimental.pallas import tpu_sc as plsc

EPS_ = 1e-5
NC_, NS_, L_ = 2, 16, 16
BS_, C_, N_ = 16, 64, 32768
NH_ = N_ // 2
NB1_ = 512
CAP_ = 17440


def _sort_body(split_hbm, idxout_hbm, scores_v, kka, kkb, kia, kib, hist,
               b2s):
    wid = lax.axis_index("s") * NC_ + lax.axis_index("c")
    lane = jnp.arange(L_, dtype=jnp.int32)
    zeros16 = jnp.zeros((L_,), jnp.int32)
    ones16 = jnp.ones((L_,), jnp.int32)
    nb1f = jnp.float32(NB1_)

    @pl.when(wid < BS_)
    def _():
        b = wid
        pltpu.sync_copy(split_hbm.at[pl.ds(b, 1)], scores_v)
        sf = scores_v.at[0]

        def zero_hist(nwords):
            def z(i, _):
                hist[pl.ds(i * L_, L_)] = zeros16
                return 0
            lax.fori_loop(0, nwords // L_, z, 0, unroll=8)

        def bin1_of(s):
            return jnp.clip((s * nb1f).astype(jnp.int32), 0, NB1_ - 1)

        zero_hist(NB1_ * L_)

        def h1(j, _):
            s = sf[pl.ds(j * L_, L_)]
            plsc.addupdate_scatter(hist, [bin1_of(s) * L_ + lane], ones16)
            return 0
        lax.fori_loop(0, N_ // L_, h1, 0, unroll=8)

        def scan1(i, carry):
            cum, b1, a1, c1 = carry
            binv = NB1_ - 1 - i
            cnt = jnp.sum(hist[pl.ds(binv * L_, L_)])
            newcum = cum + cnt
            hit = (cum < NH_) & (newcum >= NH_)
            return (newcum,
                    jnp.where(hit, binv, b1),
                    jnp.where(hit, cum, a1),
                    jnp.where(hit, cnt, c1))
        _, b1, a1, c1 = lax.fori_loop(
            0, NB1_, scan1,
            (jnp.int32(0), jnp.int32(0), jnp.int32(0), jnp.int32(0)),
            unroll=4)
        b1f = b1.astype(jnp.float32)

        def bin2_of(s):
            t = s * nb1f - b1f
            return jnp.clip((t * nb1f).astype(jnp.int32), 0, NB1_ - 1)

        b2s[0] = jnp.int32(0)

        @pl.when(a1 + c1 > CAP_ - L_)
        def _():
            zero_hist(NB1_ * L_)

            def h2(j, _):
                s = sf[pl.ds(j * L_, L_)]
                m = bin1_of(s) == b1
                plsc.addupdate_scatter(hist, [bin2_of(s) * L_ + lane],
                                       ones16, mask=m)
                return 0
            lax.fori_loop(0, N_ // L_, h2, 0, unroll=8)

            def scan2(i, carry):
                cum, b2 = carry
                binv = NB1_ - 1 - i
                cnt = jnp.sum(hist[pl.ds(binv * L_, L_)])
                newcum = cum + cnt
                hit = (a1 + cum < NH_) & (a1 + newcum >= NH_)
                return newcum, jnp.where(hit, binv, b2)
            _, b2v = lax.fori_loop(0, NB1_, scan2,
                                   (jnp.int32(0), jnp.int32(0)), unroll=4)
            b2s[0] = b2v

        b2 = b2s[0]

        def comp(j, w):
            s = sf[pl.ds(j * L_, L_)]
            key = ~plsc.bitcast(s, jnp.int32)
            idxv = j * L_ + lane
            bb1 = bin1_of(s)
            keep = (bb1 > b1) | ((bb1 == b1) & (bin2_of(s) >= b2))
            plsc.store_compressed(kka.at[pl.ds(w, L_)], key, mask=keep)
            plsc.store_compressed(kia.at[pl.ds(w, L_)], idxv, mask=keep)
            return w + jnp.sum(keep.astype(jnp.int32))
        kcnt = lax.fori_loop(0, N_ // L_, comp, jnp.int32(0), unroll=4)
        minus1 = jnp.full((L_,), -1, jnp.int32)

        def padp(t, _):
            w = kcnt + t * L_

            @pl.when(w < CAP_)
            def _():
                kka[pl.ds(w, L_)] = minus1
                kia[pl.ds(w, L_)] = zeros16
            return 0
        lax.fori_loop(0, (CAP_ - NH_) // L_ + 1, padp, 0, unroll=4)
        chunk = CAP_ // L_

        HB_ = 256 * L_

        def zero_region(off):
            def z(i, _):
                hist[pl.ds(off + i * L_, L_)] = zeros16
                return 0
            lax.fori_loop(0, 256, z, 0, unroll=8)

        bufs = [(kka, kia, kkb, kib), (kkb, kib, kka, kia)]
        for p in range(4):
            src_k, src_i, dst_k, dst_i = bufs[p % 2]
            sh = jnp.int32(8 * p)
            zero_region(0)

            def hh(v, _):
                keyv = plsc.load_gather(src_k, [lane * chunk + v])
                d = lax.shift_right_logical(keyv, sh) & 255
                plsc.addupdate_scatter(hist, [d * L_ + lane], ones16)
                return 0
            lax.fori_loop(0, chunk, hh, 0, unroll=8)

            def pf(i, carry):
                h16 = hist[pl.ds(i * L_, L_)]
                exc = plsc.cumsum(h16) - h16
                hist[pl.ds(i * L_, L_)] = exc + carry
                return carry + jnp.sum(h16)
            lax.fori_loop(0, 256, pf, jnp.int32(0), unroll=4)

            def pm(v, _):
                addr = lane * chunk + v
                keyv = plsc.load_gather(src_k, [addr])
                iv = plsc.load_gather(src_i, [addr])
                d = lax.shift_right_logical(keyv, sh) & 255
                ha = d * L_ + lane
                pos = plsc.load_gather(hist, [ha])
                plsc.store_scatter(dst_k, [pos], keyv)
                plsc.store_scatter(dst_i, [pos], iv)
                plsc.addupdate_scatter(hist, [ha], ones16)
                return 0
            lax.fori_loop(0, chunk, pm, 0, unroll=4)

        pltpu.sync_copy(kia.at[pl.ds(0, NH_)],
                        idxout_hbm.at[pl.ds(b * NH_, NH_)])


def _sc_sort(split):
    return pl.kernel(
        _sort_body,
        out_type=jax.ShapeDtypeStruct((BS_ * NH_,), jnp.int32),
        mesh=plsc.VectorSubcoreMesh(core_axis_name="c", subcore_axis_name="s"),
        compiler_params=pltpu.CompilerParams(needs_layout_passes=False),
        scratch_types=[
            pltpu.VMEM((1, N_), jnp.float32),
            pltpu.VMEM((CAP_ + L_,), jnp.int32),
            pltpu.VMEM((CAP_ + L_,), jnp.int32),
            pltpu.VMEM((CAP_ + L_,), jnp.int32),
            pltpu.VMEM((CAP_ + L_,), jnp.int32),
            pltpu.VMEM((NB1_ * L_,), jnp.int32),
            pltpu.SMEM((1,), jnp.int32),
        ],
    )(split)


def _gather_body(x_hbm, split_hbm, idx_hbm, out_hbm, idx_v,
                 row_a, row_b, out_a, out_b, isem, osem):
    wid = lax.axis_index("s") * NC_ + lax.axis_index("c")
    b = wid // 2
    half = wid % 2
    pltpu.sync_copy(idx_hbm.at[pl.ds(b * NH_, NH_)], idx_v)

    rows = [row_a, row_b]
    outs = [out_a, out_b]

    def gather_into(row_v, out_v):
        row_f, out_f = row_v.at[0], out_v.at[0]

        @plsc.parallel_loop(0, NH_ // L_, 1, unroll=8)
        def _(j):
            iv = idx_v[pl.ds(j * L_, L_)]
            out_f[pl.ds(j * L_, L_)] = plsc.load_gather(row_f, [iv])

    def src_of(i):
        return x_hbm.at[b, pl.ds(half + 2 * i, 1)]

    def dst_of(i):
        return out_hbm.at[b, pl.ds(half + 2 * i, 1)]

    nrows = C_ // 2
    in_descs = [None, None]
    out_descs = [None, None]
    in_descs[0] = pltpu.async_copy(src_of(0), rows[0], isem)
    for i in range(nrows):
        pp = i % 2
        if i + 1 < nrows:
            in_descs[(i + 1) % 2] = pltpu.async_copy(
                src_of(i + 1), rows[(i + 1) % 2], isem)
        in_descs[pp].wait()
        if out_descs[pp] is not None:
            out_descs[pp].wait()
        gather_into(rows[pp], outs[pp])
        out_descs[pp] = pltpu.async_copy(outs[pp], dst_of(i), osem)
    for d in out_descs:
        if d is not None:
            d.wait()

    @pl.when(half == 0)
    def _():
        pltpu.sync_copy(split_hbm.at[pl.ds(b, 1)], row_a)
        gather_into(row_a, out_a)
        pltpu.sync_copy(out_a, out_hbm.at[b, pl.ds(C_, 1)])


def _sc_gather(x, split, idx):
    return pl.kernel(
        _gather_body,
        out_type=jax.ShapeDtypeStruct((BS_, C_ + 1, NH_), jnp.float32),
        mesh=plsc.VectorSubcoreMesh(core_axis_name="c", subcore_axis_name="s"),
        compiler_params=pltpu.CompilerParams(needs_layout_passes=False),
        scratch_types=[
            pltpu.VMEM((NH_,), jnp.int32),
            pltpu.VMEM((1, N_), jnp.float32),
            pltpu.VMEM((1, N_), jnp.float32),
            pltpu.VMEM((1, NH_), jnp.float32),
            pltpu.VMEM((1, NH_), jnp.float32),
            pltpu.SemaphoreType.DMA,
            pltpu.SemaphoreType.DMA,
        ],
    )(x, split, idx)


def kernel(x, gamma, beta, conv_w, conv_b):
    mean = jnp.mean(x, axis=(0, 2), keepdims=True)
    var = jnp.var(x, axis=(0, 2), keepdims=True)
    h = (x - mean) / jnp.sqrt(var + EPS_)
    h = h * gamma[None, :, None] + beta[None, :, None]
    h = jnp.maximum(h, 0.0)
    logits = jnp.einsum('bcn,c->bn', h, conv_w) + conv_b[0]
    split = jax.nn.sigmoid(logits)
    idx_flat = _sc_sort(split)
    return _sc_gather(x, split, idx_flat)


# speedup vs baseline: 1.3647x; 1.1301x over previous
import jax
import jax.numpy as jnp
from jax import lax
from jax.experimental import pallas as pl
from jax.experimental.pallas import tpu as pltpu
from jax.experimental.pallas import tpu_sc as plsc

EPS_ = 1e-5
NC_, NS_, L_ = 2, 16, 16
BS_, C_, N_ = 16, 64, 32768
NH_ = N_ // 2
NB1_ = 512
CAP_ = 17440


def _sort_body(split_hbm, idxout_hbm, scores_v, kka, kkb, kia, kib, hist,
               b2s):
    wid = lax.axis_index("s") * NC_ + lax.axis_index("c")
    lane = jnp.arange(L_, dtype=jnp.int32)
    zeros16 = jnp.zeros((L_,), jnp.int32)
    ones16 = jnp.ones((L_,), jnp.int32)
    nb1f = jnp.float32(NB1_)

    @pl.when(wid < BS_)
    def _():
        b = wid
        pltpu.sync_copy(split_hbm.at[pl.ds(b, 1)], scores_v)
        sf = scores_v.at[0]

        def zero_hist(nwords):
            def z(i, _):
                hist[pl.ds(i * L_, L_)] = zeros16
                return 0
            lax.fori_loop(0, nwords // L_, z, 0, unroll=8)

        def bin1_of(s):
            return jnp.clip((s * nb1f).astype(jnp.int32), 0, NB1_ - 1)

        zero_hist(NB1_ * L_)

        @plsc.parallel_loop(0, N_ // L_, 1, unroll=8)
        def _(j):
            s = sf[pl.ds(j * L_, L_)]
            plsc.addupdate_scatter(hist, [bin1_of(s) * L_ + lane], ones16)

        def scan1(i, carry):
            cum, b1, a1, c1 = carry
            binv = NB1_ - 1 - i
            cnt = jnp.sum(hist[pl.ds(binv * L_, L_)])
            newcum = cum + cnt
            hit = (cum < NH_) & (newcum >= NH_)
            return (newcum,
                    jnp.where(hit, binv, b1),
                    jnp.where(hit, cum, a1),
                    jnp.where(hit, cnt, c1))
        _, b1, a1, c1 = lax.fori_loop(
            0, NB1_, scan1,
            (jnp.int32(0), jnp.int32(0), jnp.int32(0), jnp.int32(0)),
            unroll=4)
        b1f = b1.astype(jnp.float32)

        def bin2_of(s):
            t = s * nb1f - b1f
            return jnp.clip((t * nb1f).astype(jnp.int32), 0, NB1_ - 1)

        b2s[0] = jnp.int32(0)

        @pl.when(a1 + c1 > CAP_ - L_)
        def _():
            zero_hist(NB1_ * L_)

            @plsc.parallel_loop(0, N_ // L_, 1, unroll=8)
            def _(j):
                s = sf[pl.ds(j * L_, L_)]
                m = bin1_of(s) == b1
                plsc.addupdate_scatter(hist, [bin2_of(s) * L_ + lane],
                                       ones16, mask=m)

            def scan2(i, carry):
                cum, b2 = carry
                binv = NB1_ - 1 - i
                cnt = jnp.sum(hist[pl.ds(binv * L_, L_)])
                newcum = cum + cnt
                hit = (a1 + cum < NH_) & (a1 + newcum >= NH_)
                return newcum, jnp.where(hit, binv, b2)
            _, b2v = lax.fori_loop(0, NB1_, scan2,
                                   (jnp.int32(0), jnp.int32(0)), unroll=4)
            b2s[0] = b2v

        b2 = b2s[0]

        def comp(j, w):
            s = sf[pl.ds(j * L_, L_)]
            key = ~plsc.bitcast(s, jnp.int32)
            idxv = j * L_ + lane
            bb1 = bin1_of(s)
            keep = (bb1 > b1) | ((bb1 == b1) & (bin2_of(s) >= b2))
            plsc.store_compressed(kka.at[pl.ds(w, L_)], key, mask=keep)
            plsc.store_compressed(kia.at[pl.ds(w, L_)], idxv, mask=keep)
            return w + jnp.sum(keep.astype(jnp.int32))
        kcnt = lax.fori_loop(0, N_ // L_, comp, jnp.int32(0), unroll=4)
        minus1 = jnp.full((L_,), -1, jnp.int32)

        def padp(t, _):
            w = kcnt + t * L_

            @pl.when(w < CAP_)
            def _():
                kka[pl.ds(w, L_)] = minus1
                kia[pl.ds(w, L_)] = zeros16
            return 0
        lax.fori_loop(0, (CAP_ - NH_) // L_ + 1, padp, 0, unroll=4)
        chunk = CAP_ // L_

        HB_ = 256 * L_

        def zero_region(off):
            def z(i, _):
                hist[pl.ds(off + i * L_, L_)] = zeros16
                return 0
            lax.fori_loop(0, 256, z, 0, unroll=8)

        bufs = [(kka, kia, kkb, kib), (kkb, kib, kka, kia)]
        for p in range(4):
            src_k, src_i, dst_k, dst_i = bufs[p % 2]
            sh = jnp.int32(8 * p)
            zero_region(0)

            @plsc.parallel_loop(0, chunk, 1, unroll=8)
            def _(v):
                keyv = plsc.load_gather(src_k, [lane * chunk + v])
                d = lax.shift_right_logical(keyv, sh) & 255
                plsc.addupdate_scatter(hist, [d * L_ + lane], ones16)

            def pf(i, carry):
                h16 = hist[pl.ds(i * L_, L_)]
                exc = plsc.cumsum(h16) - h16
                hist[pl.ds(i * L_, L_)] = exc + carry
                return carry + jnp.sum(h16)
            lax.fori_loop(0, 256, pf, jnp.int32(0), unroll=4)

            def pm(v, _):
                addr = lane * chunk + v
                keyv = plsc.load_gather(src_k, [addr])
                iv = plsc.load_gather(src_i, [addr])
                d = lax.shift_right_logical(keyv, sh) & 255
                ha = d * L_ + lane
                pos = plsc.load_gather(hist, [ha])
                if p < 3:
                    plsc.store_scatter(dst_k, [pos], keyv)
                plsc.store_scatter(dst_i, [pos], iv)
                plsc.addupdate_scatter(hist, [ha], ones16)
                return 0
            lax.fori_loop(0, chunk, pm, 0, unroll=4)

        pltpu.sync_copy(kia.at[pl.ds(0, NH_)],
                        idxout_hbm.at[pl.ds(b * NH_, NH_)])


def _sc_sort(split):
    return pl.kernel(
        _sort_body,
        out_type=jax.ShapeDtypeStruct((BS_ * NH_,), jnp.int32),
        mesh=plsc.VectorSubcoreMesh(core_axis_name="c", subcore_axis_name="s"),
        compiler_params=pltpu.CompilerParams(needs_layout_passes=False),
        scratch_types=[
            pltpu.VMEM((1, N_), jnp.float32),
            pltpu.VMEM((CAP_ + L_,), jnp.int32),
            pltpu.VMEM((CAP_ + L_,), jnp.int32),
            pltpu.VMEM((CAP_ + L_,), jnp.int32),
            pltpu.VMEM((CAP_ + L_,), jnp.int32),
            pltpu.VMEM((NB1_ * L_,), jnp.int32),
            pltpu.SMEM((1,), jnp.int32),
        ],
    )(split)


def _gather_body(x_hbm, split_hbm, idx_hbm, out_hbm, idx_v,
                 row_a, row_b, out_a, out_b, isem, osem):
    wid = lax.axis_index("s") * NC_ + lax.axis_index("c")
    b = wid // 2
    half = wid % 2
    pltpu.sync_copy(idx_hbm.at[pl.ds(b * NH_, NH_)], idx_v)

    rows = [row_a, row_b]
    outs = [out_a, out_b]

    def gather_into(row_v, out_v):
        row_f, out_f = row_v.at[0], out_v.at[0]

        @plsc.parallel_loop(0, NH_ // L_, 1, unroll=8)
        def _(j):
            iv = idx_v[pl.ds(j * L_, L_)]
            out_f[pl.ds(j * L_, L_)] = plsc.load_gather(row_f, [iv])

    def src_of(i):
        return x_hbm.at[b, pl.ds(half + 2 * i, 1)]

    def dst_of(i):
        return out_hbm.at[b, pl.ds(half + 2 * i, 1)]

    nrows = C_ // 2
    in_descs = [None, None]
    out_descs = [None, None]
    in_descs[0] = pltpu.async_copy(src_of(0), rows[0], isem)
    for i in range(nrows):
        pp = i % 2
        if i + 1 < nrows:
            in_descs[(i + 1) % 2] = pltpu.async_copy(
                src_of(i + 1), rows[(i + 1) % 2], isem)
        in_descs[pp].wait()
        if out_descs[pp] is not None:
            out_descs[pp].wait()
        gather_into(rows[pp], outs[pp])
        out_descs[pp] = pltpu.async_copy(outs[pp], dst_of(i), osem)
    for d in out_descs:
        if d is not None:
            d.wait()

    @pl.when(half == 0)
    def _():
        pltpu.sync_copy(split_hbm.at[pl.ds(b, 1)], row_a)
        gather_into(row_a, out_a)
        pltpu.sync_copy(out_a, out_hbm.at[b, pl.ds(C_, 1)])


def _sc_gather(x, split, idx):
    return pl.kernel(
        _gather_body,
        out_type=jax.ShapeDtypeStruct((BS_, C_ + 1, NH_), jnp.float32),
        mesh=plsc.VectorSubcoreMesh(core_axis_name="c", subcore_axis_name="s"),
        compiler_params=pltpu.CompilerParams(needs_layout_passes=False),
        scratch_types=[
            pltpu.VMEM((NH_,), jnp.int32),
            pltpu.VMEM((1, N_), jnp.float32),
            pltpu.VMEM((1, N_), jnp.float32),
            pltpu.VMEM((1, NH_), jnp.float32),
            pltpu.VMEM((1, NH_), jnp.float32),
            pltpu.SemaphoreType.DMA,
            pltpu.SemaphoreType.DMA,
        ],
    )(x, split, idx)


def kernel(x, gamma, beta, conv_w, conv_b):
    mean = jnp.mean(x, axis=(0, 2), keepdims=True)
    var = jnp.var(x, axis=(0, 2), keepdims=True)
    h = (x - mean) / jnp.sqrt(var + EPS_)
    h = h * gamma[None, :, None] + beta[None, :, None]
    h = jnp.maximum(h, 0.0)
    logits = jnp.einsum('bcn,c->bn', h, conv_w) + conv_b[0]
    split = jax.nn.sigmoid(logits)
    idx_flat = _sc_sort(split)
    return _sc_gather(x, split, idx_flat)


# speedup vs baseline: 1.4682x; 1.0759x over previous
import jax
import jax.numpy as jnp
from jax import lax
from jax.experimental import pallas as pl
from jax.experimental.pallas import tpu as pltpu
from jax.experimental.pallas import tpu_sc as plsc

EPS_ = 1e-5
NC_, NS_, L_ = 2, 16, 16
BS_, C_, N_ = 16, 64, 32768
NH_ = N_ // 2
NB1_ = 512
CAP_ = 17440


def _sort_body(split_hbm, idxout_hbm, scores_v, kka, kkb, kia, kib, hist,
               b2s):
    wid = lax.axis_index("s") * NC_ + lax.axis_index("c")
    lane = jnp.arange(L_, dtype=jnp.int32)
    zeros16 = jnp.zeros((L_,), jnp.int32)
    ones16 = jnp.ones((L_,), jnp.int32)
    nb1f = jnp.float32(NB1_)

    @pl.when(wid < BS_)
    def _():
        b = wid
        pltpu.sync_copy(split_hbm.at[pl.ds(b, 1)], scores_v)
        sf = scores_v.at[0]

        def zero_hist(nwords):
            @plsc.parallel_loop(0, nwords // L_, 1, unroll=8)
            def _(i):
                hist[pl.ds(i * L_, L_)] = zeros16

        def bin1_of(s):
            return jnp.clip((s * nb1f).astype(jnp.int32), 0, NB1_ - 1)

        zero_hist(NB1_ * L_)

        @plsc.parallel_loop(0, N_ // L_, 1, unroll=8)
        def _(j):
            s = sf[pl.ds(j * L_, L_)]
            plsc.addupdate_scatter(hist, [bin1_of(s) * L_ + lane], ones16)

        def scan1(i, carry):
            cum, b1, a1, c1 = carry
            binv = NB1_ - 1 - i
            cnt = jnp.sum(hist[pl.ds(binv * L_, L_)])
            newcum = cum + cnt
            hit = (cum < NH_) & (newcum >= NH_)
            return (newcum,
                    jnp.where(hit, binv, b1),
                    jnp.where(hit, cum, a1),
                    jnp.where(hit, cnt, c1))
        _, b1, a1, c1 = lax.fori_loop(
            0, NB1_, scan1,
            (jnp.int32(0), jnp.int32(0), jnp.int32(0), jnp.int32(0)),
            unroll=4)
        b1f = b1.astype(jnp.float32)

        def bin2_of(s):
            t = s * nb1f - b1f
            return jnp.clip((t * nb1f).astype(jnp.int32), 0, NB1_ - 1)

        b2s[0] = jnp.int32(0)

        @pl.when(a1 + c1 > CAP_ - L_)
        def _():
            zero_hist(NB1_ * L_)

            @plsc.parallel_loop(0, N_ // L_, 1, unroll=8)
            def _(j):
                s = sf[pl.ds(j * L_, L_)]
                m = bin1_of(s) == b1
                plsc.addupdate_scatter(hist, [bin2_of(s) * L_ + lane],
                                       ones16, mask=m)

            def scan2(i, carry):
                cum, b2 = carry
                binv = NB1_ - 1 - i
                cnt = jnp.sum(hist[pl.ds(binv * L_, L_)])
                newcum = cum + cnt
                hit = (a1 + cum < NH_) & (a1 + newcum >= NH_)
                return newcum, jnp.where(hit, binv, b2)
            _, b2v = lax.fori_loop(0, NB1_, scan2,
                                   (jnp.int32(0), jnp.int32(0)), unroll=4)
            b2s[0] = b2v

        b2 = b2s[0]

        @plsc.parallel_loop(0, N_ // L_, 1, unroll=4, carry=jnp.int32(0))
        def comp(j, w):
            s = sf[pl.ds(j * L_, L_)]
            key = ~plsc.bitcast(s, jnp.int32)
            idxv = j * L_ + lane
            bb1 = bin1_of(s)
            keep = (bb1 > b1) | ((bb1 == b1) & (bin2_of(s) >= b2))
            plsc.store_compressed(kka.at[pl.ds(w, L_)], key, mask=keep)
            plsc.store_compressed(kia.at[pl.ds(w, L_)], idxv, mask=keep)
            return w + jnp.sum(keep.astype(jnp.int32))
        kcnt = comp
        minus1 = jnp.full((L_,), -1, jnp.int32)

        def padp(t, _):
            w = kcnt + t * L_

            @pl.when(w < CAP_)
            def _():
                kka[pl.ds(w, L_)] = minus1
                kia[pl.ds(w, L_)] = zeros16
            return 0
        lax.fori_loop(0, (CAP_ - NH_) // L_ + 1, padp, 0, unroll=4)
        chunk = CAP_ // L_

        HB_ = 256 * L_

        def zero_region(off):
            def z(i, _):
                hist[pl.ds(off + i * L_, L_)] = zeros16
                return 0
            lax.fori_loop(0, 256, z, 0, unroll=8)

        bufs = [(kka, kia, kkb, kib), (kkb, kib, kka, kia)]
        for p in range(4):
            src_k, src_i, dst_k, dst_i = bufs[p % 2]
            sh = jnp.int32(8 * p)
            zero_region(0)

            @plsc.parallel_loop(0, chunk, 1, unroll=8)
            def _(v):
                keyv = plsc.load_gather(src_k, [lane * chunk + v])
                d = lax.shift_right_logical(keyv, sh) & 255
                plsc.addupdate_scatter(hist, [d * L_ + lane], ones16)

            @plsc.parallel_loop(0, 256, 1, unroll=4, carry=jnp.int32(0))
            def pf(i, carry):
                h16 = hist[pl.ds(i * L_, L_)]
                exc = plsc.cumsum(h16) - h16
                hist[pl.ds(i * L_, L_)] = exc + carry
                return carry + jnp.sum(h16)

            def pm(v, _):
                addr = lane * chunk + v
                keyv = plsc.load_gather(src_k, [addr])
                iv = plsc.load_gather(src_i, [addr])
                d = lax.shift_right_logical(keyv, sh) & 255
                ha = d * L_ + lane
                pos = plsc.load_gather(hist, [ha])
                if p < 3:
                    plsc.store_scatter(dst_k, [pos], keyv)
                plsc.store_scatter(dst_i, [pos], iv)
                plsc.addupdate_scatter(hist, [ha], ones16)
                return 0
            lax.fori_loop(0, chunk, pm, 0, unroll=4)

        pltpu.sync_copy(kia.at[pl.ds(0, NH_)],
                        idxout_hbm.at[pl.ds(b * NH_, NH_)])


def _sc_sort(split):
    return pl.kernel(
        _sort_body,
        out_type=jax.ShapeDtypeStruct((BS_ * NH_,), jnp.int32),
        mesh=plsc.VectorSubcoreMesh(core_axis_name="c", subcore_axis_name="s"),
        compiler_params=pltpu.CompilerParams(needs_layout_passes=False),
        scratch_types=[
            pltpu.VMEM((1, N_), jnp.float32),
            pltpu.VMEM((CAP_ + L_,), jnp.int32),
            pltpu.VMEM((CAP_ + L_,), jnp.int32),
            pltpu.VMEM((CAP_ + L_,), jnp.int32),
            pltpu.VMEM((CAP_ + L_,), jnp.int32),
            pltpu.VMEM((NB1_ * L_,), jnp.int32),
            pltpu.SMEM((1,), jnp.int32),
        ],
    )(split)


def _gather_body(x_hbm, split_hbm, idx_hbm, out_hbm, idx_v,
                 row_a, row_b, out_a, out_b, isem, osem):
    wid = lax.axis_index("s") * NC_ + lax.axis_index("c")
    b = wid // 2
    half = wid % 2
    pltpu.sync_copy(idx_hbm.at[pl.ds(b * NH_, NH_)], idx_v)

    rows = [row_a, row_b]
    outs = [out_a, out_b]

    def gather_into(row_v, out_v):
        row_f, out_f = row_v.at[0], out_v.at[0]

        @plsc.parallel_loop(0, NH_ // L_, 1, unroll=8)
        def _(j):
            iv = idx_v[pl.ds(j * L_, L_)]
            out_f[pl.ds(j * L_, L_)] = plsc.load_gather(row_f, [iv])

    def src_of(i):
        return x_hbm.at[b, pl.ds(half + 2 * i, 1)]

    def dst_of(i):
        return out_hbm.at[b, pl.ds(half + 2 * i, 1)]

    nrows = C_ // 2
    in_descs = [None, None]
    out_descs = [None, None]
    in_descs[0] = pltpu.async_copy(src_of(0), rows[0], isem)
    for i in range(nrows):
        pp = i % 2
        if i + 1 < nrows:
            in_descs[(i + 1) % 2] = pltpu.async_copy(
                src_of(i + 1), rows[(i + 1) % 2], isem)
        in_descs[pp].wait()
        if out_descs[pp] is not None:
            out_descs[pp].wait()
        gather_into(rows[pp], outs[pp])
        out_descs[pp] = pltpu.async_copy(outs[pp], dst_of(i), osem)
    for d in out_descs:
        if d is not None:
            d.wait()

    @pl.when(half == 0)
    def _():
        pltpu.sync_copy(split_hbm.at[pl.ds(b, 1)], row_a)
        gather_into(row_a, out_a)
        pltpu.sync_copy(out_a, out_hbm.at[b, pl.ds(C_, 1)])


def _sc_gather(x, split, idx):
    return pl.kernel(
        _gather_body,
        out_type=jax.ShapeDtypeStruct((BS_, C_ + 1, NH_), jnp.float32),
        mesh=plsc.VectorSubcoreMesh(core_axis_name="c", subcore_axis_name="s"),
        compiler_params=pltpu.CompilerParams(needs_layout_passes=False),
        scratch_types=[
            pltpu.VMEM((NH_,), jnp.int32),
            pltpu.VMEM((1, N_), jnp.float32),
            pltpu.VMEM((1, N_), jnp.float32),
            pltpu.VMEM((1, NH_), jnp.float32),
            pltpu.VMEM((1, NH_), jnp.float32),
            pltpu.SemaphoreType.DMA,
            pltpu.SemaphoreType.DMA,
        ],
    )(x, split, idx)


def kernel(x, gamma, beta, conv_w, conv_b):
    mean = jnp.mean(x, axis=(0, 2), keepdims=True)
    var = jnp.var(x, axis=(0, 2), keepdims=True)
    h = (x - mean) / jnp.sqrt(var + EPS_)
    h = h * gamma[None, :, None] + beta[None, :, None]
    h = jnp.maximum(h, 0.0)
    logits = jnp.einsum('bcn,c->bn', h, conv_w) + conv_b[0]
    split = jax.nn.sigmoid(logits)
    idx_flat = _sc_sort(split)
    return _sc_gather(x, split, idx_flat)
